# baseline (speedup 1.0000x reference)
import functools

import jax
import jax.numpy as jnp
from jax import lax
from jax.experimental import pallas as pl
from jax.experimental.pallas import tpu as pltpu
from jax.experimental.pallas import tpu_sc as plsc

B = 16384
EMB = 16
ROWS_PER_LINE = 8
LINE = ROWS_PER_LINE * EMB

_NC = 2
_NS = 16
_NW = _NC * _NS
_BPW = B // _NW
_CHUNK = 128
_NCHUNK = _BPW // _CHUNK
_NBUF = 4


def _sc_gather_body(uq_hbm, vq_hbm, wq_hbm, hq_hbm, ug_out, vg_out,
                    uqv, vqv, bufs, gsems, wsems):
    wid = lax.axis_index("s") * _NC + lax.axis_index("c")
    base = wid * _BPW
    row0 = wid * _NCHUNK
    cu = pltpu.async_copy(uq_hbm.at[pl.ds(row0, _NCHUNK)], uqv, gsems[0])
    cv = pltpu.async_copy(vq_hbm.at[pl.ds(row0, _NCHUNK)], vqv, gsems[1])
    cu.wait()
    cv.wait()
    jobs = []
    for c in range(_NCHUNK):
        jobs.append((uqv.at[c], wq_hbm, ug_out.at[pl.ds(base + c * _CHUNK, _CHUNK)]))
        jobs.append((vqv.at[c], hq_hbm, vg_out.at[pl.ds(base + c * _CHUNK, _CHUNK)]))
    gd = [None] * len(jobs)
    wd = [None] * len(jobs)
    for i, (qr, tab, dst) in enumerate(jobs):
        s = i % _NBUF
        if i >= _NBUF:
            wd[i - _NBUF].wait()
        gd[i] = pltpu.async_copy(tab.at[qr], bufs[s], gsems[s])
        if i >= 1:
            prev = i - 1
            gd[prev].wait()
            wd[prev] = pltpu.async_copy(bufs[prev % _NBUF], jobs[prev][2],
                                        wsems[prev % _NBUF])
    last = len(jobs) - 1
    gd[last].wait()
    wd[last] = pltpu.async_copy(bufs[last % _NBUF], jobs[last][2],
                                wsems[last % _NBUF])
    for i in range(max(0, len(jobs) - _NBUF), len(jobs)):
        wd[i].wait()


_sc_gather = functools.partial(
    pl.kernel,
    mesh=plsc.VectorSubcoreMesh(core_axis_name="c", subcore_axis_name="s"),
    out_type=[jax.ShapeDtypeStruct((B, LINE), jnp.float32),
              jax.ShapeDtypeStruct((B, LINE), jnp.float32)],
    scratch_types=[pltpu.VMEM((_NCHUNK, _CHUNK), jnp.int32),
                   pltpu.VMEM((_NCHUNK, _CHUNK), jnp.int32),
                   [pltpu.VMEM((_CHUNK, LINE), jnp.float32)
                    for _ in range(_NBUF)],
                   [pltpu.SemaphoreType.DMA for _ in range(_NBUF)],
                   [pltpu.SemaphoreType.DMA for _ in range(_NBUF)]],
)(_sc_gather_body)


_BLK = 2048


def _mlp_body(ug_ref, vg_ref, ur_ref, vr_ref, w1_ref, w2_ref, b2_ref, o_ref):
    w1 = w1_ref[...]
    ur = ur_ref[...]
    vr = vr_ref[...]
    ug = ug_ref[...]
    vg = vg_ref[...]
    u = jnp.zeros((_BLK, EMB), jnp.float32)
    v = jnp.zeros((_BLK, EMB), jnp.float32)
    for k in range(ROWS_PER_LINE):
        u = u + jnp.where(ur == k, ug[:, k * EMB:(k + 1) * EMB], 0.0)
        v = v + jnp.where(vr == k, vg[:, k * EMB:(k + 1) * EMB], 0.0)
    h = lax.dot_general(u, w1[:, :EMB], (((1,), (1,)), ((), ())),
                        preferred_element_type=jnp.float32)
    h = h + lax.dot_general(v, w1[:, EMB:], (((1,), (1,)), ((), ())),
                            preferred_element_type=jnp.float32)
    h = jnp.maximum(h, 0.0)
    logit = jnp.sum(h * w2_ref[...], axis=1) + b2_ref[0]
    o_ref[...] = 1.0 / (1.0 + jnp.exp(-logit))


def _mlp(ug, vg, ur, vr, w1, w2, b2):
    return pl.pallas_call(
        _mlp_body,
        grid=(B // _BLK,),
        in_specs=[
            pl.BlockSpec((_BLK, LINE), lambda i: (i, 0)),
            pl.BlockSpec((_BLK, LINE), lambda i: (i, 0)),
            pl.BlockSpec((_BLK, 1), lambda i: (i, 0)),
            pl.BlockSpec((_BLK, 1), lambda i: (i, 0)),
            pl.BlockSpec((EMB, 2 * EMB), lambda i: (0, 0)),
            pl.BlockSpec((1, EMB), lambda i: (0, 0)),
            pl.BlockSpec(memory_space=pltpu.SMEM),
        ],
        out_specs=pl.BlockSpec((_BLK,), lambda i: (i,)),
        out_shape=jax.ShapeDtypeStruct((B,), jnp.float32),
    )(ug, vg, ur, vr, w1, w2, b2)


def kernel(x, W, H, W1, W2, b2):
    uidx = x[:, 0]
    vidx = x[:, 1]
    uq = (uidx // ROWS_PER_LINE).reshape(_NW * _NCHUNK, _CHUNK)
    vq = (vidx // ROWS_PER_LINE).reshape(_NW * _NCHUNK, _CHUNK)
    ur = (uidx % ROWS_PER_LINE)[:, None]
    vr = (vidx % ROWS_PER_LINE)[:, None]
    wq = W.reshape(-1, LINE)
    hq = H.reshape(-1, LINE)
    ug, vg = _sc_gather(uq, vq, wq, hq)
    return _mlp(ug, vg, ur, vr, W1, W2, b2)


# speedup vs baseline: 1.0873x; 1.0873x over previous
import functools

import jax
import jax.numpy as jnp
from jax import lax
from jax.experimental import pallas as pl
from jax.experimental.pallas import tpu as pltpu
from jax.experimental.pallas import tpu_sc as plsc

B = 16384
EMB = 16

_NC = 2
_NS = 16
_NW = _NC * _NS
_BPW = B // _NW
_CHUNK = 128
_NCHUNK = _BPW // _CHUNK


def _sc_gather_body(uidx_hbm, vidx_hbm, w_hbm, h_hbm, u_out, v_out,
                    uidx_v, vidx_v, urows, vrows, semu, semv):
    wid = lax.axis_index("s") * _NC + lax.axis_index("c")
    base = wid * _BPW
    row0 = wid * _NCHUNK
    cu = pltpu.async_copy(uidx_hbm.at[pl.ds(row0, _NCHUNK)], uidx_v, semu)
    cv = pltpu.async_copy(vidx_hbm.at[pl.ds(row0, _NCHUNK)], vidx_v, semv)
    cu.wait()
    gu = [pltpu.async_copy(w_hbm.at[uidx_v.at[j]],
                           urows.at[pl.ds(j * _CHUNK, _CHUNK)], semu)
          for j in range(_NCHUNK)]
    cv.wait()
    gv = [pltpu.async_copy(h_hbm.at[vidx_v.at[j]],
                           vrows.at[pl.ds(j * _CHUNK, _CHUNK)], semv)
          for j in range(_NCHUNK)]
    for g in gu:
        g.wait()
    ou = pltpu.async_copy(urows, u_out.at[pl.ds(base, _BPW)], semu)
    for g in gv:
        g.wait()
    ov = pltpu.async_copy(vrows, v_out.at[pl.ds(base, _BPW)], semv)
    ou.wait()
    ov.wait()


_sc_gather = functools.partial(
    pl.kernel,
    mesh=plsc.VectorSubcoreMesh(core_axis_name="c", subcore_axis_name="s"),
    out_type=[jax.ShapeDtypeStruct((B, EMB), jnp.float32),
              jax.ShapeDtypeStruct((B, EMB), jnp.float32)],
    scratch_types=[pltpu.VMEM((_NCHUNK, _CHUNK), jnp.int32),
                   pltpu.VMEM((_NCHUNK, _CHUNK), jnp.int32),
                   pltpu.VMEM((_BPW, EMB), jnp.float32),
                   pltpu.VMEM((_BPW, EMB), jnp.float32),
                   pltpu.SemaphoreType.DMA,
                   pltpu.SemaphoreType.DMA],
    compiler_params=pltpu.CompilerParams(use_tc_tiling_on_sc=False),
)(_sc_gather_body)


_BLK = 2048


def _mlp_body(u_ref, v_ref, w1_ref, w2_ref, b2_ref, o_ref):
    u = u_ref[...]
    v = v_ref[...]
    w1 = w1_ref[...]
    h = lax.dot_general(u, w1[:, :EMB], (((1,), (1,)), ((), ())),
                        preferred_element_type=jnp.float32)
    h = h + lax.dot_general(v, w1[:, EMB:], (((1,), (1,)), ((), ())),
                            preferred_element_type=jnp.float32)
    h = jnp.maximum(h, 0.0)
    logit = jnp.sum(h * w2_ref[...], axis=1) + b2_ref[0]
    o_ref[...] = 1.0 / (1.0 + jnp.exp(-logit))


def _mlp(u, v, w1, w2, b2):
    return pl.pallas_call(
        _mlp_body,
        grid=(B // _BLK,),
        in_specs=[
            pl.BlockSpec((_BLK, EMB), lambda i: (i, 0)),
            pl.BlockSpec((_BLK, EMB), lambda i: (i, 0)),
            pl.BlockSpec((EMB, 2 * EMB), lambda i: (0, 0)),
            pl.BlockSpec((1, EMB), lambda i: (0, 0)),
            pl.BlockSpec(memory_space=pltpu.SMEM),
        ],
        out_specs=pl.BlockSpec((_BLK,), lambda i: (i,)),
        out_shape=jax.ShapeDtypeStruct((B,), jnp.float32),
    )(u, v, w1, w2, b2)


def kernel(x, W, H, W1, W2, b2):
    uidx = x[:, 0].reshape(_NW * _NCHUNK, _CHUNK)
    vidx = x[:, 1].reshape(_NW * _NCHUNK, _CHUNK)
    w_lin = lax.optimization_barrier(W.reshape(-1)).reshape(W.shape)
    h_lin = lax.optimization_barrier(H.reshape(-1)).reshape(H.shape)
    U, V = _sc_gather(uidx, vidx, w_lin, h_lin)
    return _mlp(U, V, W1, W2, b2)


# speedup vs baseline: 1.3633x; 1.2539x over previous
import functools

import jax
import jax.numpy as jnp
from jax import lax
from jax.experimental import pallas as pl
from jax.experimental.pallas import tpu as pltpu
from jax.experimental.pallas import tpu_sc as plsc

B = 16384
EMB = 16
NROW = 1000000
ROWS_PER_LINE = 8
LINE = ROWS_PER_LINE * EMB

_NC = 2
_NS = 16
_NW = _NC * _NS
_BPW = B // _NW
_CHUNK = 128
_NCHUNK = _BPW // _CHUNK


_TCOLS = 8192
_TSUB = _TCOLS // ROWS_PER_LINE
_TGRID = -(-NROW // _TCOLS)
NLINE = _TGRID * _TSUB


def _transpose_body(wt_ref, o_ref):
    w = wt_ref[...]
    parts = [w[:, s * _TSUB:(s + 1) * _TSUB].T for s in range(ROWS_PER_LINE)]
    o_ref[...] = jnp.concatenate(parts, axis=1)


def _to_lines(wt):
    return pl.pallas_call(
        _transpose_body,
        grid=(_TGRID,),
        in_specs=[pl.BlockSpec((EMB, _TCOLS), lambda i: (0, i))],
        out_specs=pl.BlockSpec((_TSUB, LINE), lambda i: (i, 0)),
        out_shape=jax.ShapeDtypeStruct((NLINE, LINE), jnp.float32),
    )(wt)


def _sc_gather_body(uq_hbm, vq_hbm, wq_hbm, hq_hbm, ug_out, vg_out,
                    uqv, vqv, bufs, gsems, wsems):
    wid = lax.axis_index("s") * _NC + lax.axis_index("c")
    base = wid * _BPW
    row0 = wid * _NCHUNK
    cu = pltpu.async_copy(uq_hbm.at[pl.ds(row0, _NCHUNK)], uqv, gsems[0])
    cv = pltpu.async_copy(vq_hbm.at[pl.ds(row0, _NCHUNK)], vqv, gsems[1])
    cu.wait()
    cv.wait()
    jobs = []
    for c in range(_NCHUNK):
        jobs.append((uqv.at[c], wq_hbm, ug_out.at[pl.ds(base + c * _CHUNK, _CHUNK)]))
        jobs.append((vqv.at[c], hq_hbm, vg_out.at[pl.ds(base + c * _CHUNK, _CHUNK)]))
    nbuf = len(bufs)
    gd = [None] * len(jobs)
    wd = [None] * len(jobs)
    for i, (qr, tab, dst) in enumerate(jobs):
        s = i % nbuf
        if i >= nbuf:
            wd[i - nbuf].wait()
        gd[i] = pltpu.async_copy(tab.at[qr], bufs[s], gsems[s])
        if i >= 1:
            prev = i - 1
            gd[prev].wait()
            wd[prev] = pltpu.async_copy(bufs[prev % nbuf], jobs[prev][2],
                                        wsems[prev % nbuf])
    last = len(jobs) - 1
    gd[last].wait()
    wd[last] = pltpu.async_copy(bufs[last % nbuf], jobs[last][2],
                                wsems[last % nbuf])
    for i in range(max(0, len(jobs) - nbuf), len(jobs)):
        wd[i].wait()


_NBUF = 4

_sc_gather = functools.partial(
    pl.kernel,
    mesh=plsc.VectorSubcoreMesh(core_axis_name="c", subcore_axis_name="s"),
    out_type=[jax.ShapeDtypeStruct((B, LINE), jnp.float32),
              jax.ShapeDtypeStruct((B, LINE), jnp.float32)],
    scratch_types=[pltpu.VMEM((_NCHUNK, _CHUNK), jnp.int32),
                   pltpu.VMEM((_NCHUNK, _CHUNK), jnp.int32),
                   [pltpu.VMEM((_CHUNK, LINE), jnp.float32)
                    for _ in range(_NBUF)],
                   [pltpu.SemaphoreType.DMA for _ in range(_NBUF)],
                   [pltpu.SemaphoreType.DMA for _ in range(_NBUF)]],
)(_sc_gather_body)


_BLK = 2048


def _mlp_body(ug_ref, vg_ref, ur_ref, vr_ref, w1_ref, w2_ref, b2_ref, o_ref):
    w1 = w1_ref[...]
    ur = ur_ref[...]
    vr = vr_ref[...]
    ug = ug_ref[...]
    vg = vg_ref[...]
    u = jnp.zeros((_BLK, EMB), jnp.float32)
    v = jnp.zeros((_BLK, EMB), jnp.float32)
    for k in range(ROWS_PER_LINE):
        u = u + jnp.where(ur == k, ug[:, k * EMB:(k + 1) * EMB], 0.0)
        v = v + jnp.where(vr == k, vg[:, k * EMB:(k + 1) * EMB], 0.0)
    h = lax.dot_general(u, w1[:, :EMB], (((1,), (1,)), ((), ())),
                        preferred_element_type=jnp.float32)
    h = h + lax.dot_general(v, w1[:, EMB:], (((1,), (1,)), ((), ())),
                            preferred_element_type=jnp.float32)
    h = jnp.maximum(h, 0.0)
    logit = jnp.sum(h * w2_ref[...], axis=1) + b2_ref[0]
    o_ref[...] = 1.0 / (1.0 + jnp.exp(-logit))


def _mlp(ug, vg, ur, vr, w1, w2, b2):
    return pl.pallas_call(
        _mlp_body,
        grid=(B // _BLK,),
        in_specs=[
            pl.BlockSpec((_BLK, LINE), lambda i: (i, 0)),
            pl.BlockSpec((_BLK, LINE), lambda i: (i, 0)),
            pl.BlockSpec((_BLK, 1), lambda i: (i, 0)),
            pl.BlockSpec((_BLK, 1), lambda i: (i, 0)),
            pl.BlockSpec((EMB, 2 * EMB), lambda i: (0, 0)),
            pl.BlockSpec((1, EMB), lambda i: (0, 0)),
            pl.BlockSpec(memory_space=pltpu.SMEM),
        ],
        out_specs=pl.BlockSpec((_BLK,), lambda i: (i,)),
        out_shape=jax.ShapeDtypeStruct((B,), jnp.float32),
    )(ug, vg, ur, vr, w1, w2, b2)


def kernel(x, W, H, W1, W2, b2):
    uidx = x[:, 0]
    vidx = x[:, 1]
    uline = (uidx // _TCOLS) * _TSUB + (uidx % _TSUB)
    vline = (vidx // _TCOLS) * _TSUB + (vidx % _TSUB)
    uq = uline.reshape(_NW * _NCHUNK, _CHUNK)
    vq = vline.reshape(_NW * _NCHUNK, _CHUNK)
    ur = ((uidx % _TCOLS) // _TSUB)[:, None]
    vr = ((vidx % _TCOLS) // _TSUB)[:, None]
    wq = _to_lines(W.T)
    hq = _to_lines(H.T)
    ug, vg = _sc_gather(uq, vq, wq, hq)
    return _mlp(ug, vg, ur, vr, W1, W2, b2)


# speedup vs baseline: 2.1318x; 1.5637x over previous
import functools

import jax
import jax.numpy as jnp
from jax import lax
from jax.experimental import pallas as pl
from jax.experimental.pallas import tpu as pltpu
from jax.experimental.pallas import tpu_sc as plsc

B = 16384
EMB = 16
NROW = 1000000
SLOTS = 8
LINE = SLOTS * EMB

_NC = 2
_NS = 16
_NW = _NC * _NS
_BPW = B // _NW
_CHUNK = 128
_NCHUNK = _BPW // _CHUNK


_TCOLS = 32768
_TSUB = _TCOLS // SLOTS
_TGRID = -(-NROW // _TCOLS)
NLINE = _TGRID * _TSUB


def _lines_body(wt_ref, gg_ref, o_ref):
    w = wt_ref[...]
    t = w.T
    acc = lax.dot_general(t[0:_TSUB], gg_ref[0:EMB, :],
                          (((1,), (0,)), ((), ())),
                          preferred_element_type=jnp.float32)
    for s in range(1, SLOTS):
        acc = acc + lax.dot_general(
            t[s * _TSUB:(s + 1) * _TSUB], gg_ref[s * EMB:(s + 1) * EMB, :],
            (((1,), (0,)), ((), ())), preferred_element_type=jnp.float32)
    o_ref[...] = acc


def _to_lines(wt, gg):
    return pl.pallas_call(
        _lines_body,
        grid=(_TGRID,),
        in_specs=[pl.BlockSpec((EMB, _TCOLS), lambda i: (0, i)),
                  pl.BlockSpec((LINE, LINE), lambda i: (0, 0))],
        out_specs=pl.BlockSpec((_TSUB, LINE), lambda i: (i, 0)),
        out_shape=jax.ShapeDtypeStruct((NLINE, LINE), jnp.float32),
    )(wt, gg)


def _sc_gather_body(uq_hbm, vq_hbm, wq_hbm, hq_hbm, ug_out, vg_out,
                    uqv, vqv, bufs, gsems, wsems):
    wid = lax.axis_index("s") * _NC + lax.axis_index("c")
    base = wid * _BPW
    row0 = wid * _NCHUNK
    cu = pltpu.async_copy(uq_hbm.at[pl.ds(row0, _NCHUNK)], uqv, gsems[0])
    cv = pltpu.async_copy(vq_hbm.at[pl.ds(row0, _NCHUNK)], vqv, gsems[1])
    cu.wait()
    cv.wait()
    jobs = []
    for c in range(_NCHUNK):
        jobs.append((uqv.at[c], wq_hbm, ug_out.at[pl.ds(base + c * _CHUNK, _CHUNK)]))
        jobs.append((vqv.at[c], hq_hbm, vg_out.at[pl.ds(base + c * _CHUNK, _CHUNK)]))
    nbuf = len(bufs)
    gd = [None] * len(jobs)
    wd = [None] * len(jobs)
    for i, (qr, tab, dst) in enumerate(jobs):
        s = i % nbuf
        if i >= nbuf:
            wd[i - nbuf].wait()
        gd[i] = pltpu.async_copy(tab.at[qr], bufs[s], gsems[s])
        if i >= 1:
            prev = i - 1
            gd[prev].wait()
            wd[prev] = pltpu.async_copy(bufs[prev % nbuf], jobs[prev][2],
                                        wsems[prev % nbuf])
    last = len(jobs) - 1
    gd[last].wait()
    wd[last] = pltpu.async_copy(bufs[last % nbuf], jobs[last][2],
                                wsems[last % nbuf])
    for i in range(max(0, len(jobs) - nbuf), len(jobs)):
        wd[i].wait()


_NBUF = 4

_sc_gather = functools.partial(
    pl.kernel,
    mesh=plsc.VectorSubcoreMesh(core_axis_name="c", subcore_axis_name="s"),
    out_type=[jax.ShapeDtypeStruct((B, LINE), jnp.float32),
              jax.ShapeDtypeStruct((B, LINE), jnp.float32)],
    scratch_types=[pltpu.VMEM((_NCHUNK, _CHUNK), jnp.int32),
                   pltpu.VMEM((_NCHUNK, _CHUNK), jnp.int32),
                   [pltpu.VMEM((_CHUNK, LINE), jnp.float32)
                    for _ in range(_NBUF)],
                   [pltpu.SemaphoreType.DMA for _ in range(_NBUF)],
                   [pltpu.SemaphoreType.DMA for _ in range(_NBUF)]],
)(_sc_gather_body)


_BLK = 2048


def _mlp_body(ug_ref, vg_ref, ur_ref, vr_ref, w2_ref, b2_ref, o_ref):
    ur = ur_ref[...]
    vr = vr_ref[...]
    ug = ug_ref[...]
    vg = vg_ref[...]
    h = jnp.zeros((_BLK, EMB), jnp.float32)
    for k in range(SLOTS):
        h = h + jnp.where(ur == k, ug[:, k * EMB:(k + 1) * EMB], 0.0)
        h = h + jnp.where(vr == k, vg[:, k * EMB:(k + 1) * EMB], 0.0)
    h = jnp.maximum(h, 0.0)
    logit = jnp.sum(h * w2_ref[...], axis=1) + b2_ref[0]
    o_ref[...] = 1.0 / (1.0 + jnp.exp(-logit))


def _mlp(ug, vg, ur, vr, w2, b2):
    return pl.pallas_call(
        _mlp_body,
        grid=(B // _BLK,),
        in_specs=[
            pl.BlockSpec((_BLK, LINE), lambda i: (i, 0)),
            pl.BlockSpec((_BLK, LINE), lambda i: (i, 0)),
            pl.BlockSpec((_BLK, 1), lambda i: (i, 0)),
            pl.BlockSpec((_BLK, 1), lambda i: (i, 0)),
            pl.BlockSpec((1, EMB), lambda i: (0, 0)),
            pl.BlockSpec(memory_space=pltpu.SMEM),
        ],
        out_specs=pl.BlockSpec((_BLK,), lambda i: (i,)),
        out_shape=jax.ShapeDtypeStruct((B,), jnp.float32),
    )(ug, vg, ur, vr, w2, b2)


def kernel(x, W, H, W1, W2, b2):
    uidx = x[:, 0]
    vidx = x[:, 1]
    uline = (uidx // _TCOLS) * _TSUB + (uidx % _TSUB)
    vline = (vidx // _TCOLS) * _TSUB + (vidx % _TSUB)
    uq = uline.reshape(_NW * _NCHUNK, _CHUNK)
    vq = vline.reshape(_NW * _NCHUNK, _CHUNK)
    ur = ((uidx % _TCOLS) // _TSUB)[:, None]
    vr = ((vidx % _TCOLS) // _TSUB)[:, None]
    eye8 = jnp.eye(SLOTS, dtype=jnp.float32)
    wq = _to_lines(W.T, jnp.kron(eye8, W1[:, :EMB].T))
    hq = _to_lines(H.T, jnp.kron(eye8, W1[:, EMB:].T))
    ug, vg = _sc_gather(uq, vq, wq, hq)
    return _mlp(ug, vg, ur, vr, W2, b2)


# speedup vs baseline: 3.1526x; 1.4788x over previous
import functools

import jax
import jax.numpy as jnp
from jax import lax
from jax.experimental import pallas as pl
from jax.experimental.pallas import tpu as pltpu
from jax.experimental.pallas import tpu_sc as plsc

B = 16384
EMB = 16
NROW = 1000000
SLOTS = 8
LINE = SLOTS * EMB

_NC = 2
_NS = 16
_NW = _NC * _NS
_BPW = B // _NW
_CHUNK = 128
_NCHUNK = _BPW // _CHUNK


_TCOLS = 32768
_TSUB = _TCOLS // SLOTS
_TGRID = -(-NROW // _TCOLS)
NLINE = _TGRID * _TSUB


def _one_table(w, gg_ref):
    t = w.astype(jnp.bfloat16).T
    acc = lax.dot_general(t[0:_TSUB], gg_ref[0:EMB, :],
                          (((1,), (0,)), ((), ())),
                          preferred_element_type=jnp.float32)
    for s in range(1, SLOTS):
        acc = acc + lax.dot_general(
            t[s * _TSUB:(s + 1) * _TSUB], gg_ref[s * EMB:(s + 1) * EMB, :],
            (((1,), (0,)), ((), ())), preferred_element_type=jnp.float32)
    return acc


def _lines_body(wt_ref, ht_ref, ggu_ref, ggv_ref, ou_ref, ov_ref):
    ou_ref[...] = _one_table(wt_ref[...], ggu_ref)
    ov_ref[...] = _one_table(ht_ref[...], ggv_ref)


def _to_lines(wt, ht, ggu, ggv):
    return pl.pallas_call(
        _lines_body,
        grid=(_TGRID,),
        in_specs=[pl.BlockSpec((EMB, _TCOLS), lambda i: (0, i)),
                  pl.BlockSpec((EMB, _TCOLS), lambda i: (0, i)),
                  pl.BlockSpec((LINE, LINE), lambda i: (0, 0)),
                  pl.BlockSpec((LINE, LINE), lambda i: (0, 0))],
        out_specs=[pl.BlockSpec((_TSUB, LINE), lambda i: (i, 0)),
                   pl.BlockSpec((_TSUB, LINE), lambda i: (i, 0))],
        out_shape=[jax.ShapeDtypeStruct((NLINE, LINE), jnp.float32),
                   jax.ShapeDtypeStruct((NLINE, LINE), jnp.float32)],
    )(wt, ht, ggu, ggv)


def _sc_gather_body(uq_hbm, vq_hbm, wq_hbm, hq_hbm, ug_out, vg_out,
                    uqv, vqv, bufs, gsems, wsems):
    wid = lax.axis_index("s") * _NC + lax.axis_index("c")
    base = wid * _BPW
    row0 = wid * _NCHUNK
    cu = pltpu.async_copy(uq_hbm.at[pl.ds(row0, _NCHUNK)], uqv, gsems[0])
    cv = pltpu.async_copy(vq_hbm.at[pl.ds(row0, _NCHUNK)], vqv, gsems[1])
    cu.wait()
    cv.wait()
    jobs = []
    for c in range(_NCHUNK):
        jobs.append((uqv.at[c], wq_hbm, ug_out.at[pl.ds(base + c * _CHUNK, _CHUNK)]))
        jobs.append((vqv.at[c], hq_hbm, vg_out.at[pl.ds(base + c * _CHUNK, _CHUNK)]))
    nbuf = len(bufs)
    gd = [None] * len(jobs)
    wd = [None] * len(jobs)
    for i, (qr, tab, dst) in enumerate(jobs):
        s = i % nbuf
        if i >= nbuf:
            wd[i - nbuf].wait()
        gd[i] = pltpu.async_copy(tab.at[qr], bufs[s], gsems[s])
        if i >= 1:
            prev = i - 1
            gd[prev].wait()
            wd[prev] = pltpu.async_copy(bufs[prev % nbuf], jobs[prev][2],
                                        wsems[prev % nbuf])
    last = len(jobs) - 1
    gd[last].wait()
    wd[last] = pltpu.async_copy(bufs[last % nbuf], jobs[last][2],
                                wsems[last % nbuf])
    for i in range(max(0, len(jobs) - nbuf), len(jobs)):
        wd[i].wait()


_NBUF = 4

_sc_gather = functools.partial(
    pl.kernel,
    mesh=plsc.VectorSubcoreMesh(core_axis_name="c", subcore_axis_name="s"),
    out_type=[jax.ShapeDtypeStruct((B, LINE), jnp.float32),
              jax.ShapeDtypeStruct((B, LINE), jnp.float32)],
    scratch_types=[pltpu.VMEM((_NCHUNK, _CHUNK), jnp.int32),
                   pltpu.VMEM((_NCHUNK, _CHUNK), jnp.int32),
                   [pltpu.VMEM((_CHUNK, LINE), jnp.float32)
                    for _ in range(_NBUF)],
                   [pltpu.SemaphoreType.DMA for _ in range(_NBUF)],
                   [pltpu.SemaphoreType.DMA for _ in range(_NBUF)]],
)(_sc_gather_body)


_BLK = 2048


def _mlp_body(ug_ref, vg_ref, ur_ref, vr_ref, w2_ref, b2_ref, o_ref):
    ur = ur_ref[...]
    vr = vr_ref[...]
    ug = ug_ref[...]
    vg = vg_ref[...]
    h = jnp.zeros((_BLK, EMB), jnp.float32)
    for k in range(SLOTS):
        h = h + jnp.where(ur == k, ug[:, k * EMB:(k + 1) * EMB], 0.0)
        h = h + jnp.where(vr == k, vg[:, k * EMB:(k + 1) * EMB], 0.0)
    h = jnp.maximum(h, 0.0)
    logit = jnp.sum(h * w2_ref[...], axis=1) + b2_ref[0]
    o_ref[...] = 1.0 / (1.0 + jnp.exp(-logit))


def _mlp(ug, vg, ur, vr, w2, b2):
    return pl.pallas_call(
        _mlp_body,
        grid=(B // _BLK,),
        in_specs=[
            pl.BlockSpec((_BLK, LINE), lambda i: (i, 0)),
            pl.BlockSpec((_BLK, LINE), lambda i: (i, 0)),
            pl.BlockSpec((_BLK, 1), lambda i: (i, 0)),
            pl.BlockSpec((_BLK, 1), lambda i: (i, 0)),
            pl.BlockSpec((1, EMB), lambda i: (0, 0)),
            pl.BlockSpec(memory_space=pltpu.SMEM),
        ],
        out_specs=pl.BlockSpec((_BLK,), lambda i: (i,)),
        out_shape=jax.ShapeDtypeStruct((B,), jnp.float32),
    )(ug, vg, ur, vr, w2, b2)


def kernel(x, W, H, W1, W2, b2):
    uidx = x[:, 0]
    vidx = x[:, 1]
    uline = (uidx // _TCOLS) * _TSUB + (uidx % _TSUB)
    vline = (vidx // _TCOLS) * _TSUB + (vidx % _TSUB)
    uq = uline.reshape(_NW * _NCHUNK, _CHUNK)
    vq = vline.reshape(_NW * _NCHUNK, _CHUNK)
    ur = ((uidx % _TCOLS) // _TSUB)[:, None]
    vr = ((vidx % _TCOLS) // _TSUB)[:, None]
    eye8 = jnp.eye(SLOTS, dtype=jnp.float32)
    ggu = jnp.kron(eye8, W1[:, :EMB].T).astype(jnp.bfloat16)
    ggv = jnp.kron(eye8, W1[:, EMB:].T).astype(jnp.bfloat16)
    wq, hq = _to_lines(W.T, H.T, ggu, ggv)
    ug, vg = _sc_gather(uq, vq, wq, hq)
    return _mlp(ug, vg, ur, vr, W2, b2)
